# baseline (speedup 1.0000x reference)
import jax
import jax.numpy as jnp
from jax.experimental import pallas as pl
from jax.experimental.pallas import tpu as pltpu

_IN = 4096
_HID = 8192
_K = 409
_LR = 0.001
_HALF = _HID // 2

_BH = 1024
_BR = 512
_BU = 256


def _enc_body(w_ref, xc_ref, b_ref, o_ref):
    acc = jax.lax.dot_general(
        w_ref[...], xc_ref[...], (((1,), (0,)), ((), ())),
        preferred_element_type=jnp.float32)
    o_ref[...] = jnp.maximum(acc + b_ref[...], 0.0)


def _kth_bits(h_row):
    bits = jax.lax.bitcast_convert_type(h_row, jnp.int32)

    def body(_, carry):
        lo, hi = carry
        mid = lo + jnp.right_shift(hi - lo, 1)
        cnt = jnp.sum(jnp.where(bits >= mid, 1, 0), axis=1, keepdims=True)
        take = cnt >= _K
        return jnp.where(take, mid, lo), jnp.where(take, hi, mid)

    lo0 = jnp.zeros((1, 1), jnp.int32)
    hi0 = jnp.full((1, 1), 0x7F800001, jnp.int32)
    lo, _ = jax.lax.fori_loop(0, 31, body, (lo0, hi0))
    return lo


def _mask(h, kth):
    bits = jax.lax.bitcast_convert_type(h, jnp.int32)
    return jnp.where(bits >= kth, h, 0.0)


def _pass1_body(hr_ref, hc_ref, s_ref, y_ref, ss_ref, kth_ref, hsq_ref,
                kth_s):
    r = pl.program_id(1)

    @pl.when(r == 0)
    def _():
        kth = _kth_bits(hr_ref[...])
        kth_s[...] = kth
        kth_ref[...] = kth
        hs = _mask(hr_ref[...], kth)
        hsq_ref[...] = jnp.sum(hs * hs, axis=1, keepdims=True)
        y_ref[...] = jnp.zeros_like(y_ref)
        ss_ref[...] = jnp.zeros_like(ss_ref)

    kth = kth_s[...]
    y_acc = jnp.zeros((1, _HID), jnp.float32)
    q_acc = jnp.zeros((1, _HID), jnp.float32)
    for k in range(_BR // 128):
        sk = s_ref[k * 128:(k + 1) * 128, :]
        hk = _mask(hc_ref[k * 128:(k + 1) * 128, :], kth)
        y_acc = y_acc + jnp.sum(sk * hk, axis=0, keepdims=True)
        q_acc = q_acc + jnp.sum(sk * sk, axis=0, keepdims=True)
    y_ref[...] += y_acc.reshape(1, 1, _HID)
    ss_ref[...] += q_acc.reshape(1, 1, _HID)


def _upd_body(s_ref, hc_ref, hr_ref, kth_ref, y_ref, ss_ref, st_ref, wc_ref,
              bc_ref, hsq_ref, o_ref, lg_ref, ns_ref, inv_s, hrs_s):
    r = pl.program_id(1)
    kth = kth_ref[...]

    @pl.when(r == 0)
    def _():
        hs_row = _mask(hr_ref[...], kth)
        hrs_s[...] = hs_row * _LR
        y = jnp.sum(y_ref[...].reshape(2, _HID), axis=0, keepdims=True)
        ns = jnp.tanh(y + st_ref[...])
        ns_ref[...] = ns
        lg = jax.lax.dot_general(
            ns, wc_ref[...], (((1,), (1,)), ((), ())),
            preferred_element_type=jnp.float32)
        lg_ref[...] = lg + bc_ref[...]
        dot_yh = jnp.sum(y * hs_row, axis=1, keepdims=True)
        ss2 = ss_ref[...].reshape(2, _HID)
        ssq = jnp.sum(jnp.sum(ss2, axis=1, keepdims=True), axis=0,
                      keepdims=True)
        hsq = hsq_ref[...]
        norm2 = ssq + (2.0 * _LR) * dot_yh + (_LR * _LR) * hsq * hsq
        inv_s[...] = jax.lax.rsqrt(norm2)

    inv = inv_s[...]
    hrs = hrs_s[...]
    for k in range(_BU // 128):
        sk = s_ref[k * 128:(k + 1) * 128, :]
        hk = _mask(hc_ref[k * 128:(k + 1) * 128, :], kth)
        o_ref[k * 128:(k + 1) * 128, :] = (sk + hk * hrs) * inv


def _encode(W, xc, bc):
    nb = _HID // _BH // 2
    return pl.pallas_call(
        _enc_body,
        grid=(2, nb),
        in_specs=[
            pl.BlockSpec((_BH, _IN), lambda c, r: (c * nb + r, 0)),
            pl.BlockSpec((_IN, 1), lambda c, r: (0, 0)),
            pl.BlockSpec((_BH, 1), lambda c, r: (c * nb + r, 0)),
        ],
        out_specs=pl.BlockSpec((_BH, 1), lambda c, r: (c * nb + r, 0)),
        out_shape=jax.ShapeDtypeStruct((_HID, 1), jnp.float32),
        compiler_params=pltpu.CompilerParams(
            dimension_semantics=("parallel", "arbitrary"),
            vmem_limit_bytes=48 * 1024 * 1024),
        name="bdh_encode",
    )(W, xc, bc)


def _pass1(h_row, h_col, S):
    nr = _HID // _BR // 2
    zero = lambda c, r: (0, 0)
    return pl.pallas_call(
        _pass1_body,
        grid=(2, nr),
        in_specs=[
            pl.BlockSpec((1, _HID), zero),
            pl.BlockSpec((_BR, 1), lambda c, r: (c * nr + r, 0)),
            pl.BlockSpec((_BR, _HID), lambda c, r: (c * nr + r, 0)),
        ],
        out_specs=[
            pl.BlockSpec((1, 1, _HID), lambda c, r: (c, 0, 0)),
            pl.BlockSpec((1, 1, _HID), lambda c, r: (c, 0, 0)),
            pl.BlockSpec((1, 1), zero),
            pl.BlockSpec((1, 1), zero),
        ],
        out_shape=[jax.ShapeDtypeStruct((2, 1, _HID), jnp.float32),
                   jax.ShapeDtypeStruct((2, 1, _HID), jnp.float32),
                   jax.ShapeDtypeStruct((1, 1), jnp.int32),
                   jax.ShapeDtypeStruct((1, 1), jnp.float32)],
        scratch_shapes=[pltpu.VMEM((1, 1), jnp.int32)],
        compiler_params=pltpu.CompilerParams(
            dimension_semantics=("parallel", "arbitrary"),
            vmem_limit_bytes=48 * 1024 * 1024),
        name="bdh_pass1",
    )(h_row, h_col, S)


def _update(S, h_col, h_row, kth, y2, ss2, state, W_cls, bc2, hsq):
    nr = _HID // _BU // 2
    zero = lambda c, r: (0, 0)
    zero3 = lambda c, r: (0, 0, 0)
    return pl.pallas_call(
        _upd_body,
        grid=(2, nr),
        in_specs=[
            pl.BlockSpec((_BU, _HID), lambda c, r: (c * nr + r, 0)),
            pl.BlockSpec((_BU, 1), lambda c, r: (c * nr + r, 0)),
            pl.BlockSpec((1, _HID), zero),
            pl.BlockSpec((1, 1), zero),
            pl.BlockSpec((2, 1, _HID), zero3),
            pl.BlockSpec((2, 1, _HID), zero3),
            pl.BlockSpec((1, _HID), zero),
            pl.BlockSpec((2, _HID), zero),
            pl.BlockSpec((1, 2), zero),
            pl.BlockSpec((1, 1), zero),
        ],
        out_specs=[
            pl.BlockSpec((_BU, _HID), lambda c, r: (c * nr + r, 0)),
            pl.BlockSpec((1, 2), zero),
            pl.BlockSpec((1, _HID), zero),
        ],
        out_shape=[jax.ShapeDtypeStruct((_HID, _HID), jnp.float32),
                   jax.ShapeDtypeStruct((1, 2), jnp.float32),
                   jax.ShapeDtypeStruct((1, _HID), jnp.float32)],
        scratch_shapes=[pltpu.VMEM((1, 1), jnp.float32),
                        pltpu.VMEM((1, _HID), jnp.float32)],
        compiler_params=pltpu.CompilerParams(
            dimension_semantics=("parallel", "arbitrary"),
            vmem_limit_bytes=48 * 1024 * 1024),
        name="bdh_update",
    )(S, h_col, h_row, kth, y2, ss2, state, W_cls, bc2, hsq)


def kernel(x, W_enc, b_enc, synapses, state, W_cls, b_cls):
    xc = x.reshape(_IN, 1)
    bc = b_enc.reshape(_HID, 1)
    h_col = _encode(W_enc, xc, bc)
    h_row = h_col.reshape(1, _HID)
    y2, ss2, kth, hsq = _pass1(h_row, h_col, synapses)
    new_synapses, logits, new_state = _update(
        synapses, h_col, h_row, kth, y2, ss2, state, W_cls,
        b_cls.reshape(1, 2), hsq)
    return logits, new_state, new_synapses


# speedup vs baseline: 1.0166x; 1.0166x over previous
import jax
import jax.numpy as jnp
from jax.experimental import pallas as pl
from jax.experimental.pallas import tpu as pltpu

_IN = 4096
_HID = 8192
_K = 409
_LR = 0.001

_BH = 512
_BR = 512
_BU = 256


def _enc_body(w_ref, xc_ref, b_ref, o_ref):
    acc = jax.lax.dot_general(
        w_ref[...], xc_ref[...], (((1,), (0,)), ((), ())),
        preferred_element_type=jnp.float32)
    o_ref[...] = jnp.maximum(acc + b_ref[...], 0.0)


def _topk_body(h_ref, hs_ref, hsq_ref):
    h = h_ref[...]
    bits = jax.lax.bitcast_convert_type(h, jnp.int32)

    def body(_, carry):
        lo, hi = carry
        mid = lo + jnp.right_shift(hi - lo, 1)
        cnt = jnp.sum(jnp.where(bits >= mid, 1, 0), axis=1, keepdims=True)
        take = cnt >= _K
        return jnp.where(take, mid, lo), jnp.where(take, hi, mid)

    lo0 = jnp.zeros((1, 1), jnp.int32)
    hi0 = jnp.full((1, 1), 0x7F800001, jnp.int32)
    lo, _ = jax.lax.fori_loop(0, 31, body, (lo0, hi0))
    hs = jnp.where(bits >= lo, h, 0.0)
    hs_ref[...] = hs
    hsq_ref[...] = jnp.sum(hs * hs, axis=1, keepdims=True)


def _pass1_body(hc_ref, s_ref, y_ref, ss_ref):
    r = pl.program_id(1)

    @pl.when(r == 0)
    def _():
        y_ref[...] = jnp.zeros_like(y_ref)
        ss_ref[...] = jnp.zeros_like(ss_ref)

    y_acc = jnp.zeros((1, _HID), jnp.float32)
    q_acc = jnp.zeros((1, _HID), jnp.float32)
    for k in range(_BR // 128):
        sk = s_ref[k * 128:(k + 1) * 128, :]
        hk = hc_ref[k * 128:(k + 1) * 128, :]
        y_acc = y_acc + jnp.sum(sk * hk, axis=0, keepdims=True)
        q_acc = q_acc + jnp.sum(sk * sk, axis=0, keepdims=True)
    y_ref[...] += y_acc.reshape(1, 1, _HID)
    ss_ref[...] += q_acc.reshape(1, 1, _HID)


def _upd_body(s_ref, hc_ref, hr_ref, y_ref, ss_ref, st_ref, wc_ref, bc_ref,
              hsq_ref, o_ref, lg_ref, ns_ref, inv_s):
    r = pl.program_id(1)

    @pl.when(r == 0)
    def _():
        y = jnp.sum(y_ref[...].reshape(2, _HID), axis=0, keepdims=True)
        ns = jnp.tanh(y + st_ref[...])
        ns_ref[...] = ns
        lg = jax.lax.dot_general(
            ns, wc_ref[...], (((1,), (1,)), ((), ())),
            preferred_element_type=jnp.float32)
        lg_ref[...] = lg + bc_ref[...]
        dot_yh = jnp.sum(y * hr_ref[...], axis=1, keepdims=True)
        ss2 = ss_ref[...].reshape(2, _HID)
        ssq = jnp.sum(jnp.sum(ss2, axis=1, keepdims=True), axis=0,
                      keepdims=True)
        hsq = hsq_ref[...]
        norm2 = ssq + (2.0 * _LR) * dot_yh + (_LR * _LR) * hsq * hsq
        inv_s[...] = jax.lax.rsqrt(norm2)

    inv = inv_s[...]
    hrs = hr_ref[...] * _LR
    for k in range(_BU // 128):
        sk = s_ref[k * 128:(k + 1) * 128, :]
        hk = hc_ref[k * 128:(k + 1) * 128, :]
        o_ref[k * 128:(k + 1) * 128, :] = (sk + hk * hrs) * inv


def _encode(W, xc, bc):
    nb = _HID // _BH // 2
    return pl.pallas_call(
        _enc_body,
        grid=(2, nb),
        in_specs=[
            pl.BlockSpec((_BH, _IN), lambda c, r: (c * nb + r, 0)),
            pl.BlockSpec((_IN, 1), lambda c, r: (0, 0)),
            pl.BlockSpec((_BH, 1), lambda c, r: (c * nb + r, 0)),
        ],
        out_specs=pl.BlockSpec((_BH, 1), lambda c, r: (c * nb + r, 0)),
        out_shape=jax.ShapeDtypeStruct((_HID, 1), jnp.float32),
        compiler_params=pltpu.CompilerParams(
            dimension_semantics=("parallel", "arbitrary"),
            vmem_limit_bytes=48 * 1024 * 1024),
        name="bdh_encode",
    )(W, xc, bc)


def _sparsify(h_row):
    return pl.pallas_call(
        _topk_body,
        out_shape=(jax.ShapeDtypeStruct((1, _HID), jnp.float32),
                   jax.ShapeDtypeStruct((1, 1), jnp.float32)),
        name="bdh_topk",
    )(h_row)


def _pass1(hs_col, S):
    nr = _HID // _BR // 2
    return pl.pallas_call(
        _pass1_body,
        grid=(2, nr),
        in_specs=[
            pl.BlockSpec((_BR, 1), lambda c, r: (c * nr + r, 0)),
            pl.BlockSpec((_BR, _HID), lambda c, r: (c * nr + r, 0)),
        ],
        out_specs=[
            pl.BlockSpec((1, 1, _HID), lambda c, r: (c, 0, 0)),
            pl.BlockSpec((1, 1, _HID), lambda c, r: (c, 0, 0)),
        ],
        out_shape=[jax.ShapeDtypeStruct((2, 1, _HID), jnp.float32),
                   jax.ShapeDtypeStruct((2, 1, _HID), jnp.float32)],
        compiler_params=pltpu.CompilerParams(
            dimension_semantics=("parallel", "arbitrary"),
            vmem_limit_bytes=48 * 1024 * 1024),
        name="bdh_pass1",
    )(hs_col, S)


def _update(S, hs_col, hs_row, y2, ss2, state, W_cls, bc2, hsq):
    nr = _HID // _BU // 2
    zero = lambda c, r: (0, 0)
    zero3 = lambda c, r: (0, 0, 0)
    return pl.pallas_call(
        _upd_body,
        grid=(2, nr),
        in_specs=[
            pl.BlockSpec((_BU, _HID), lambda c, r: (c * nr + r, 0)),
            pl.BlockSpec((_BU, 1), lambda c, r: (c * nr + r, 0)),
            pl.BlockSpec((1, _HID), zero),
            pl.BlockSpec((2, 1, _HID), zero3),
            pl.BlockSpec((2, 1, _HID), zero3),
            pl.BlockSpec((1, _HID), zero),
            pl.BlockSpec((2, _HID), zero),
            pl.BlockSpec((1, 2), zero),
            pl.BlockSpec((1, 1), zero),
        ],
        out_specs=[
            pl.BlockSpec((_BU, _HID), lambda c, r: (c * nr + r, 0)),
            pl.BlockSpec((1, 2), zero),
            pl.BlockSpec((1, _HID), zero),
        ],
        out_shape=[jax.ShapeDtypeStruct((_HID, _HID), jnp.float32),
                   jax.ShapeDtypeStruct((1, 2), jnp.float32),
                   jax.ShapeDtypeStruct((1, _HID), jnp.float32)],
        scratch_shapes=[pltpu.VMEM((1, 1), jnp.float32)],
        compiler_params=pltpu.CompilerParams(
            dimension_semantics=("parallel", "arbitrary"),
            vmem_limit_bytes=48 * 1024 * 1024),
        name="bdh_update",
    )(S, hs_col, hs_row, y2, ss2, state, W_cls, bc2, hsq)


def kernel(x, W_enc, b_enc, synapses, state, W_cls, b_cls):
    xc = x.reshape(_IN, 1)
    bc = b_enc.reshape(_HID, 1)
    h_col = _encode(W_enc, xc, bc)
    h_row = h_col.reshape(1, _HID)
    hs_row, hsq = _sparsify(h_row)
    hs_col = hs_row.reshape(_HID, 1)
    y2, ss2 = _pass1(hs_col, synapses)
    new_synapses, logits, new_state = _update(
        synapses, hs_col, hs_row, y2, ss2, state, W_cls,
        b_cls.reshape(1, 2), hsq)
    return logits, new_state, new_synapses
